# baseline (speedup 1.0000x reference)
import jax
import jax.numpy as jnp
from jax.experimental import pallas as pl


N = 10000
NHID = 128
NEMBED = 128
NCLASS = 40
BH = 200
BM = 2 * BH


def _fused_body(adj0_ref, adj1_ref, xi_ref, xf_ref, w1t_ref, w2t_ref, wmt_ref,
                b_ref, out_ref):
    xf = xf_ref[...]
    w1t = w1t_ref[...]
    w2t = w2t_ref[...]
    wmt = wmt_ref[...]
    b = b_ref[...]

    def half(adj_blk, xi, out_slice):
        deg = jnp.sum(adj_blk, axis=1, keepdims=True)
        neigh = jnp.dot(adj_blk, xf,
                        preferred_element_type=jnp.float32)
        neigh = neigh / (deg + 1.0)
        h = (jnp.dot(xi, w1t, preferred_element_type=jnp.float32) +
             jnp.dot(neigh, w2t, preferred_element_type=jnp.float32))
        h = jnp.maximum(h, 0.0)
        out_ref[out_slice, :] = (
            jnp.dot(h, wmt, preferred_element_type=jnp.float32) + b)

    half(adj0_ref[...], xi_ref[:BH, :], pl.ds(0, BH))
    half(adj1_ref[...], xi_ref[BH:, :], pl.ds(BH, BH))


@jax.jit
def kernel(x, adj, W_proj, W_mlp, b_mlp):
    w1t = W_proj[:, :NHID].T
    w2t = W_proj[:, NHID:].T
    wmt = W_mlp.T
    b2 = b_mlp.reshape(1, NCLASS)

    grid = (N // BM,)
    out = pl.pallas_call(
        _fused_body,
        grid=grid,
        in_specs=[
            pl.BlockSpec((BH, N), lambda i: (2 * i, 0)),
            pl.BlockSpec((BH, N), lambda i: (2 * i + 1, 0)),
            pl.BlockSpec((BM, NHID), lambda i: (i, 0)),
            pl.BlockSpec((N, NHID), lambda i: (0, 0)),
            pl.BlockSpec((NHID, NEMBED), lambda i: (0, 0)),
            pl.BlockSpec((NHID, NEMBED), lambda i: (0, 0)),
            pl.BlockSpec((NEMBED, NCLASS), lambda i: (0, 0)),
            pl.BlockSpec((1, NCLASS), lambda i: (0, 0)),
        ],
        out_specs=pl.BlockSpec((BM, NCLASS), lambda i: (i, 0)),
        out_shape=jax.ShapeDtypeStruct((N, NCLASS), jnp.float32),
    )(adj, adj, x, x, w1t, w2t, wmt, b2)
    return out


# speedup vs baseline: 1.1327x; 1.1327x over previous
import jax
import jax.numpy as jnp
from jax.experimental import pallas as pl


N = 10000
NHID = 128
NEMBED = 128
NCLASS = 40
BM = 400


def _fused_body(adj_ref, xf_ref, w1t_ref, w2t_ref, wmt_ref, b_ref, out_ref):
    i = pl.program_id(0)
    adj_blk = adj_ref[...]
    deg = jnp.sum(adj_blk, axis=1, keepdims=True)
    neigh = jnp.dot(adj_blk, xf_ref[...],
                    preferred_element_type=jnp.float32)
    neigh = neigh / (deg + 1.0)
    xi = xf_ref[pl.ds(i * BM, BM), :]
    h = (jnp.dot(xi, w1t_ref[...], preferred_element_type=jnp.float32) +
         jnp.dot(neigh, w2t_ref[...], preferred_element_type=jnp.float32))
    h = jnp.maximum(h, 0.0)
    out_ref[...] = (jnp.dot(h, wmt_ref[...],
                            preferred_element_type=jnp.float32) +
                    b_ref[...])


@jax.jit
def kernel(x, adj, W_proj, W_mlp, b_mlp):
    w1t = W_proj[:, :NHID].T
    w2t = W_proj[:, NHID:].T
    wmt = W_mlp.T
    b2 = b_mlp.reshape(1, NCLASS)

    grid = (N // BM,)
    out = pl.pallas_call(
        _fused_body,
        grid=grid,
        in_specs=[
            pl.BlockSpec((BM, N), lambda i: (i, 0)),
            pl.BlockSpec((N, NHID), lambda i: (0, 0)),
            pl.BlockSpec((NHID, NEMBED), lambda i: (0, 0)),
            pl.BlockSpec((NHID, NEMBED), lambda i: (0, 0)),
            pl.BlockSpec((NEMBED, NCLASS), lambda i: (0, 0)),
            pl.BlockSpec((1, NCLASS), lambda i: (0, 0)),
        ],
        out_specs=pl.BlockSpec((BM, NCLASS), lambda i: (i, 0)),
        out_shape=jax.ShapeDtypeStruct((N, NCLASS), jnp.float32),
    )(adj, x, w1t, w2t, wmt, b2)
    return out
